# baseline (speedup 1.0000x reference)
import jax
import jax.numpy as jnp
from jax.experimental import pallas as pl

N_TOKENS = 16384
EMBED = 2048
N_EXPERTS = 64
TOP_K = 8
BLK = 2048


def _router_kernel(x0_ref, x1_ref, x2_ref, x3_ref, w_ref, b_ref, probs_ref, idx_ref):
    w = w_ref[...]
    parts = [
        jax.lax.dot_general(
            w, x_ref[...], (((1,), (1,)), ((), ())),
            preferred_element_type=jnp.float32,
            precision=jax.lax.Precision.DEFAULT,
        )
        for x_ref in (x0_ref, x1_ref, x2_ref, x3_ref)
    ]
    lt = jnp.concatenate(parts, axis=1) + b_ref[...]

    iota0 = jax.lax.broadcasted_iota(jnp.int32, lt.shape, 0)
    neg = jnp.float32(-jnp.inf)
    cur = lt
    idx_rows = []
    for _ in range(TOP_K):
        m = jnp.max(cur, axis=0, keepdims=True)
        idx = jnp.min(jnp.where(cur == m, iota0, N_EXPERTS), axis=0, keepdims=True)
        cur = jnp.where(iota0 == idx, neg, cur)
        idx_rows.append(idx)
    idx_ref[...] = jnp.concatenate(idx_rows, axis=0).T

    selected = cur == neg
    mx = jnp.max(jnp.where(selected, lt, neg), axis=0, keepdims=True)
    e = jnp.where(selected, jnp.exp(lt - mx), 0.0)
    probs_ref[...] = (e / jnp.sum(e, axis=0, keepdims=True)).T


@jax.jit
def kernel(inputs, W, b):
    b2 = b.reshape(N_EXPERTS, 1)
    probs, idx = pl.pallas_call(
        _router_kernel,
        grid=(N_TOKENS // BLK,),
        in_specs=[
            pl.BlockSpec((BLK // 4, EMBED), lambda i: (4 * i, 0)),
            pl.BlockSpec((BLK // 4, EMBED), lambda i: (4 * i + 1, 0)),
            pl.BlockSpec((BLK // 4, EMBED), lambda i: (4 * i + 2, 0)),
            pl.BlockSpec((BLK // 4, EMBED), lambda i: (4 * i + 3, 0)),
            pl.BlockSpec((N_EXPERTS, EMBED), lambda i: (0, 0)),
            pl.BlockSpec((N_EXPERTS, 1), lambda i: (0, 0)),
        ],
        out_specs=[
            pl.BlockSpec((BLK, N_EXPERTS), lambda i: (i, 0)),
            pl.BlockSpec((BLK, TOP_K), lambda i: (i, 0)),
        ],
        out_shape=[
            jax.ShapeDtypeStruct((N_TOKENS, N_EXPERTS), jnp.float32),
            jax.ShapeDtypeStruct((N_TOKENS, TOP_K), jnp.int32),
        ],
    )(inputs, inputs, inputs, inputs, W, b2)
    return (probs, idx)
